# Initial kernel scaffold; baseline (speedup 1.0000x reference)
import jax
import jax.numpy as jnp
from jax.experimental import pallas as pl


def kernel(wav):
    raise NotImplementedError("write your pallas kernel here")


# speedup vs baseline: 3.6646x; 3.6646x over previous
import jax
import jax.numpy as jnp
from jax import lax
from jax.experimental import pallas as pl
from jax.experimental.pallas import tpu as pltpu
from jax.experimental.pallas import tpu_sc as plsc

_K = 2
_K_CORRECTION = 128

_C = 16
_L = 1048576
_SL = _L // _K - _K_CORRECTION
_CH = 896
_NCHUNK = _SL // _CH
_NW = 32
_ROUNDS = -(-_NCHUNK // _NW)
_GRPS = _CH // 16
_UMAX = (_ROUNDS + 2 + 1) // 2


def _sc_body(wav_hbm, lidx_hbm, out1_hbm, out2_hbm,
             in_b, idx_b, o1_b, o2_b,
             in_sem0, in_sem1, out_sem0, out_sem1):
    wid = lax.axis_index("s") * 2 + lax.axis_index("c")
    in_sems = (in_sem0, in_sem1)
    out_sems = (out_sem0, out_sem1)

    def in_copies(t, b):
        ci = wid + _NW * t
        col0 = pl.multiple_of(ci * (2 * _CH), 2 * _CH)
        idx0 = pl.multiple_of(ci * _CH, _CH)
        return (
            pltpu.make_async_copy(wav_hbm.at[:, pl.ds(col0, 2 * _CH)],
                                  in_b.at[b], in_sems[b]),
            pltpu.make_async_copy(lidx_hbm.at[pl.ds(idx0, _CH)],
                                  idx_b.at[b], in_sems[b]),
        )

    def out_copies(t, b):
        ci = wid + _NW * t
        idx0 = pl.multiple_of(ci * _CH, _CH)
        return (
            pltpu.make_async_copy(o1_b.at[b], out1_hbm.at[:, pl.ds(idx0, _CH)],
                                  out_sems[b]),
            pltpu.make_async_copy(o2_b.at[b], out2_hbm.at[:, pl.ds(idx0, _CH)],
                                  out_sems[b]),
        )

    def compute(b):
        for c in range(_C):
            cc = jnp.full((16,), c, jnp.int32)

            def g_body(g, carry, cc=cc, c=c):
                iv = idx_b[b, pl.ds(g * 16, 16)]
                w1 = plsc.load_gather(in_b.at[b], [cc, iv])
                w2 = plsc.load_gather(in_b.at[b], [cc, iv ^ 1])
                o1_b[b, c, pl.ds(g * 16, 16)] = w1
                o2_b[b, c, pl.ds(g * 16, 16)] = w2
                return carry

            lax.fori_loop(0, _GRPS, g_body, 0, unroll=2)

    @pl.when(wid < _NCHUNK)
    def _():
        for cp in in_copies(0, 0):
            cp.start()

    def u_body(u, carry):
        for b in (0, 1):
            t = 2 * u + b
            ci = wid + _NW * t

            @pl.when(ci < _NCHUNK)
            def _(t=t, b=b):
                for cp in in_copies(t, b):
                    cp.wait()

            @pl.when(wid + _NW * (t + 1) < _NCHUNK)
            def _(t=t, b=b):
                for cp in in_copies(t + 1, 1 - b):
                    cp.start()

            @pl.when((t >= 2) & (wid + _NW * (t - 2) < _NCHUNK))
            def _(t=t, b=b):
                for cp in out_copies(t - 2, b):
                    cp.wait()

            @pl.when(ci < _NCHUNK)
            def _(t=t, b=b):
                compute(b)
                for cp in out_copies(t, b):
                    cp.start()

        return carry

    lax.fori_loop(0, _UMAX, u_body, 0)


_sc_call = pl.kernel(
    _sc_body,
    out_type=(
        jax.ShapeDtypeStruct((_C, _SL), jnp.float32),
        jax.ShapeDtypeStruct((_C, _SL), jnp.float32),
    ),
    mesh=plsc.VectorSubcoreMesh(core_axis_name="c", subcore_axis_name="s"),
    scratch_types=[
        pltpu.VMEM((2, _C, 2 * _CH), jnp.float32),
        pltpu.VMEM((2, _CH), jnp.int32),
        pltpu.VMEM((2, _C, _CH), jnp.float32),
        pltpu.VMEM((2, _C, _CH), jnp.float32),
        pltpu.SemaphoreType.DMA,
        pltpu.SemaphoreType.DMA,
        pltpu.SemaphoreType.DMA,
        pltpu.SemaphoreType.DMA,
    ],
    compiler_params=pltpu.CompilerParams(needs_layout_passes=False),
)


def kernel(wav):
    rkey = jax.random.fold_in(jax.random.key(0), 1)
    r = jax.random.randint(rkey, (_SL,), 0, _K)
    lidx = ((jnp.arange(_SL, dtype=jnp.int32) % _CH) * 2 + r).astype(jnp.int32)
    return _sc_call(wav, lidx)


# speedup vs baseline: 7.2801x; 1.9866x over previous
import jax
import jax.numpy as jnp
from jax import lax
from jax.experimental import pallas as pl
from jax.experimental.pallas import tpu as pltpu
from jax.experimental.pallas import tpu_sc as plsc

_K = 2
_K_CORRECTION = 128

_C = 16
_L = 1048576
_SL = _L // _K - _K_CORRECTION
_CH = 896
_NCHUNK = _SL // _CH
_NW = 32
_ROUNDS = -(-_NCHUNK // _NW)
_GRPS = _CH // 16
_UMAX = (_ROUNDS + 2 + 1) // 2


def _sc_body(wav_hbm, lidx_hbm, out1_hbm, out2_hbm,
             in_b, idx_b, o1_b, o2_b,
             in_sem0, in_sem1, out_sem0, out_sem1):
    wid = lax.axis_index("s") * 2 + lax.axis_index("c")
    in_sems = (in_sem0, in_sem1)
    out_sems = (out_sem0, out_sem1)

    def in_copies(t, b):
        ci = wid + _NW * t
        col0 = pl.multiple_of(ci * (2 * _CH), 2 * _CH)
        idx0 = pl.multiple_of(ci * _CH, _CH)
        return (
            pltpu.make_async_copy(wav_hbm.at[:, pl.ds(col0, 2 * _CH)],
                                  in_b.at[b], in_sems[b]),
            pltpu.make_async_copy(lidx_hbm.at[pl.ds(idx0, _CH)],
                                  idx_b.at[b], in_sems[b]),
        )

    def out_copies(t, b):
        ci = wid + _NW * t
        idx0 = pl.multiple_of(ci * _CH, _CH)
        return (
            pltpu.make_async_copy(o1_b.at[b], out1_hbm.at[:, pl.ds(idx0, _CH)],
                                  out_sems[b]),
            pltpu.make_async_copy(o2_b.at[b], out2_hbm.at[:, pl.ds(idx0, _CH)],
                                  out_sems[b]),
        )

    def compute(b):
        def g_body(g, carry):
            iv = idx_b[b, pl.ds(g * 16, 16)]
            iv2 = iv ^ 1
            for c in range(_C):
                cc = jnp.full((16,), c, jnp.int32)
                w1 = plsc.load_gather(in_b.at[b], [cc, iv])
                w2 = plsc.load_gather(in_b.at[b], [cc, iv2])
                o1_b[b, c, pl.ds(g * 16, 16)] = w1
                o2_b[b, c, pl.ds(g * 16, 16)] = w2
            return carry

        lax.fori_loop(0, _GRPS, g_body, 0, unroll=2)

    @pl.when(wid < _NCHUNK)
    def _():
        for cp in in_copies(0, 0):
            cp.start()

    def u_body(u, carry):
        for b in (0, 1):
            t = 2 * u + b
            ci = wid + _NW * t

            @pl.when(ci < _NCHUNK)
            def _(t=t, b=b):
                for cp in in_copies(t, b):
                    cp.wait()

            @pl.when(wid + _NW * (t + 1) < _NCHUNK)
            def _(t=t, b=b):
                for cp in in_copies(t + 1, 1 - b):
                    cp.start()

            @pl.when((t >= 2) & (wid + _NW * (t - 2) < _NCHUNK))
            def _(t=t, b=b):
                for cp in out_copies(t - 2, b):
                    cp.wait()

            @pl.when(ci < _NCHUNK)
            def _(t=t, b=b):
                compute(b)
                for cp in out_copies(t, b):
                    cp.start()

        return carry

    lax.fori_loop(0, _UMAX, u_body, 0)


_sc_call = pl.kernel(
    _sc_body,
    out_type=(
        jax.ShapeDtypeStruct((_C, _SL), jnp.float32),
        jax.ShapeDtypeStruct((_C, _SL), jnp.float32),
    ),
    mesh=plsc.VectorSubcoreMesh(core_axis_name="c", subcore_axis_name="s"),
    scratch_types=[
        pltpu.VMEM((2, _C, 2 * _CH), jnp.float32),
        pltpu.VMEM((2, _CH), jnp.int32),
        pltpu.VMEM((2, _C, _CH), jnp.float32),
        pltpu.VMEM((2, _C, _CH), jnp.float32),
        pltpu.SemaphoreType.DMA,
        pltpu.SemaphoreType.DMA,
        pltpu.SemaphoreType.DMA,
        pltpu.SemaphoreType.DMA,
    ],
    compiler_params=pltpu.CompilerParams(needs_layout_passes=False),
)


def kernel(wav):
    rkey = jax.random.fold_in(jax.random.key(0), 1)
    r = jax.random.randint(rkey, (_SL,), 0, _K)
    lidx = ((jnp.arange(_SL, dtype=jnp.int32) % _CH) * 2 + r).astype(jnp.int32)
    return _sc_call(wav, lidx)


# speedup vs baseline: 10.7083x; 1.4709x over previous
import jax
import jax.numpy as jnp
from jax import lax
from jax.experimental import pallas as pl
from jax.experimental.pallas import tpu as pltpu
from jax.experimental.pallas import tpu_sc as plsc

_K = 2
_K_CORRECTION = 128

_C = 16
_L = 1048576
_SL = _L // _K - _K_CORRECTION
_CH = 896
_NCHUNK = _SL // _CH
_NW = 32
_ROUNDS = -(-_NCHUNK // _NW)
_GRPS = _CH // 16
_UMAX = (_ROUNDS + 2 + 1) // 2


def _sc_body(wav_hbm, lidx_hbm, out1_hbm, out2_hbm,
             in_b, idx_b, o1_b, o2_b,
             in_sem0, in_sem1, out_sem0, out_sem1):
    wid = lax.axis_index("s") * 2 + lax.axis_index("c")
    in_sems = (in_sem0, in_sem1)
    out_sems = (out_sem0, out_sem1)

    def in_copies(t, b):
        ci = wid + _NW * t
        col0 = pl.multiple_of(ci * (2 * _CH), 2 * _CH)
        idx0 = pl.multiple_of(ci * _CH, _CH)
        return (
            pltpu.make_async_copy(wav_hbm.at[:, pl.ds(col0, 2 * _CH)],
                                  in_b.at[b], in_sems[b]),
            pltpu.make_async_copy(lidx_hbm.at[pl.ds(idx0, _CH)],
                                  idx_b.at[b], in_sems[b]),
        )

    def out_copies(t, b):
        ci = wid + _NW * t
        idx0 = pl.multiple_of(ci * _CH, _CH)
        return (
            pltpu.make_async_copy(o1_b.at[b], out1_hbm.at[:, pl.ds(idx0, _CH)],
                                  out_sems[b]),
            pltpu.make_async_copy(o2_b.at[b], out2_hbm.at[:, pl.ds(idx0, _CH)],
                                  out_sems[b]),
        )

    def compute(b):
        @plsc.parallel_loop(0, _GRPS * 16, step=16, unroll=2)
        def g_body(g0):
            iv = idx_b[b, pl.ds(g0, 16)]
            iv2 = iv ^ 1
            for c in range(_C):
                cc = jnp.full((16,), c, jnp.int32)
                w1 = plsc.load_gather(in_b.at[b], [cc, iv])
                w2 = plsc.load_gather(in_b.at[b], [cc, iv2])
                o1_b[b, c, pl.ds(g0, 16)] = w1
                o2_b[b, c, pl.ds(g0, 16)] = w2

    @pl.when(wid < _NCHUNK)
    def _():
        for cp in in_copies(0, 0):
            cp.start()

    def u_body(u, carry):
        for b in (0, 1):
            t = 2 * u + b
            ci = wid + _NW * t

            @pl.when(ci < _NCHUNK)
            def _(t=t, b=b):
                for cp in in_copies(t, b):
                    cp.wait()

            @pl.when(wid + _NW * (t + 1) < _NCHUNK)
            def _(t=t, b=b):
                for cp in in_copies(t + 1, 1 - b):
                    cp.start()

            @pl.when((t >= 2) & (wid + _NW * (t - 2) < _NCHUNK))
            def _(t=t, b=b):
                for cp in out_copies(t - 2, b):
                    cp.wait()

            @pl.when(ci < _NCHUNK)
            def _(t=t, b=b):
                compute(b)
                for cp in out_copies(t, b):
                    cp.start()

        return carry

    lax.fori_loop(0, _UMAX, u_body, 0)


_sc_call = pl.kernel(
    _sc_body,
    out_type=(
        jax.ShapeDtypeStruct((_C, _SL), jnp.float32),
        jax.ShapeDtypeStruct((_C, _SL), jnp.float32),
    ),
    mesh=plsc.VectorSubcoreMesh(core_axis_name="c", subcore_axis_name="s"),
    scratch_types=[
        pltpu.VMEM((2, _C, 2 * _CH), jnp.float32),
        pltpu.VMEM((2, _CH), jnp.int32),
        pltpu.VMEM((2, _C, _CH), jnp.float32),
        pltpu.VMEM((2, _C, _CH), jnp.float32),
        pltpu.SemaphoreType.DMA,
        pltpu.SemaphoreType.DMA,
        pltpu.SemaphoreType.DMA,
        pltpu.SemaphoreType.DMA,
    ],
    compiler_params=pltpu.CompilerParams(needs_layout_passes=False),
)


def kernel(wav):
    rkey = jax.random.fold_in(jax.random.key(0), 1)
    r = jax.random.randint(rkey, (_SL,), 0, _K)
    lidx = ((jnp.arange(_SL, dtype=jnp.int32) % _CH) * 2 + r).astype(jnp.int32)
    return _sc_call(wav, lidx)


# speedup vs baseline: 10.7261x; 1.0017x over previous
import jax
import jax.numpy as jnp
from jax import lax
from jax.experimental import pallas as pl
from jax.experimental.pallas import tpu as pltpu
from jax.experimental.pallas import tpu_sc as plsc

_K = 2
_K_CORRECTION = 128

_C = 16
_L = 1048576
_SL = _L // _K - _K_CORRECTION
_CH = 896
_NCHUNK = _SL // _CH
_NW = 32
_ROUNDS = -(-_NCHUNK // _NW)
_GRPS = _CH // 16
_UMAX = (_ROUNDS + 2 + 1) // 2


def _sc_body(wav_hbm, lidx_hbm, out1_hbm, out2_hbm,
             in_b, idx_b, o1_b, o2_b,
             in_sem0, in_sem1, out_sem0, out_sem1):
    wid = lax.axis_index("s") * 2 + lax.axis_index("c")
    in_sems = (in_sem0, in_sem1)
    out_sems = (out_sem0, out_sem1)

    def in_copies(t, b):
        ci = wid + _NW * t
        col0 = pl.multiple_of(ci * (2 * _CH), 2 * _CH)
        idx0 = pl.multiple_of(ci * _CH, _CH)
        return (
            pltpu.make_async_copy(wav_hbm.at[:, pl.ds(col0, 2 * _CH)],
                                  in_b.at[b], in_sems[b]),
            pltpu.make_async_copy(lidx_hbm.at[pl.ds(idx0, _CH)],
                                  idx_b.at[b], in_sems[b]),
        )

    def out_copies(t, b):
        ci = wid + _NW * t
        idx0 = pl.multiple_of(ci * _CH, _CH)
        return (
            pltpu.make_async_copy(o1_b.at[b], out1_hbm.at[:, pl.ds(idx0, _CH)],
                                  out_sems[b]),
            pltpu.make_async_copy(o2_b.at[b], out2_hbm.at[:, pl.ds(idx0, _CH)],
                                  out_sems[b]),
        )

    def compute(b):
        @plsc.parallel_loop(0, _GRPS * 16, step=16, unroll=4)
        def g_body(g0):
            iv = idx_b[b, pl.ds(g0, 16)]
            iv2 = iv ^ 1
            for c in range(_C):
                cc = jnp.full((16,), c, jnp.int32)
                w1 = plsc.load_gather(in_b.at[b], [cc, iv])
                w2 = plsc.load_gather(in_b.at[b], [cc, iv2])
                o1_b[b, c, pl.ds(g0, 16)] = w1
                o2_b[b, c, pl.ds(g0, 16)] = w2

    @pl.when(wid < _NCHUNK)
    def _():
        for cp in in_copies(0, 0):
            cp.start()

    def u_body(u, carry):
        for b in (0, 1):
            t = 2 * u + b
            ci = wid + _NW * t

            @pl.when(ci < _NCHUNK)
            def _(t=t, b=b):
                for cp in in_copies(t, b):
                    cp.wait()

            @pl.when(wid + _NW * (t + 1) < _NCHUNK)
            def _(t=t, b=b):
                for cp in in_copies(t + 1, 1 - b):
                    cp.start()

            @pl.when((t >= 2) & (wid + _NW * (t - 2) < _NCHUNK))
            def _(t=t, b=b):
                for cp in out_copies(t - 2, b):
                    cp.wait()

            @pl.when(ci < _NCHUNK)
            def _(t=t, b=b):
                compute(b)
                for cp in out_copies(t, b):
                    cp.start()

        return carry

    lax.fori_loop(0, _UMAX, u_body, 0)


_sc_call = pl.kernel(
    _sc_body,
    out_type=(
        jax.ShapeDtypeStruct((_C, _SL), jnp.float32),
        jax.ShapeDtypeStruct((_C, _SL), jnp.float32),
    ),
    mesh=plsc.VectorSubcoreMesh(core_axis_name="c", subcore_axis_name="s"),
    scratch_types=[
        pltpu.VMEM((2, _C, 2 * _CH), jnp.float32),
        pltpu.VMEM((2, _CH), jnp.int32),
        pltpu.VMEM((2, _C, _CH), jnp.float32),
        pltpu.VMEM((2, _C, _CH), jnp.float32),
        pltpu.SemaphoreType.DMA,
        pltpu.SemaphoreType.DMA,
        pltpu.SemaphoreType.DMA,
        pltpu.SemaphoreType.DMA,
    ],
    compiler_params=pltpu.CompilerParams(needs_layout_passes=False),
)


def kernel(wav):
    rkey = jax.random.fold_in(jax.random.key(0), 1)
    r = jax.random.randint(rkey, (_SL,), 0, _K)
    lidx = ((jnp.arange(_SL, dtype=jnp.int32) % _CH) * 2 + r).astype(jnp.int32)
    return _sc_call(wav, lidx)


# speedup vs baseline: 10.7274x; 1.0001x over previous
import jax
import jax.numpy as jnp
from jax import lax
from jax.experimental import pallas as pl
from jax.experimental.pallas import tpu as pltpu
from jax.experimental.pallas import tpu_sc as plsc

_K = 2
_K_CORRECTION = 128

_C = 16
_L = 1048576
_SL = _L // _K - _K_CORRECTION
_CH = 896
_NCHUNK = _SL // _CH
_NW = 32
_ROUNDS = -(-_NCHUNK // _NW)
_GRPS = _CH // 16
_UMAX = (_ROUNDS + 2 + 1) // 2


def _sc_body(wav_hbm, lidx_hbm, out1_hbm, out2_hbm,
             in_b, idx_b, o1_b, o2_b,
             in_sem0, in_sem1, out_sem0, out_sem1):
    wid = lax.axis_index("s") * 2 + lax.axis_index("c")
    in_sems = (in_sem0, in_sem1)
    out_sems = (out_sem0, out_sem1)

    def in_copies(t, b):
        ci = wid + _NW * t
        col0 = pl.multiple_of(ci * (2 * _CH), 2 * _CH)
        idx0 = pl.multiple_of(ci * _CH, _CH)
        return (
            pltpu.make_async_copy(wav_hbm.at[:, pl.ds(col0, 2 * _CH)],
                                  in_b.at[b], in_sems[b]),
            pltpu.make_async_copy(lidx_hbm.at[pl.ds(idx0, _CH)],
                                  idx_b.at[b], in_sems[b]),
        )

    def out_copies(t, b):
        ci = wid + _NW * t
        idx0 = pl.multiple_of(ci * _CH, _CH)
        return (
            pltpu.make_async_copy(o1_b.at[b], out1_hbm.at[:, pl.ds(idx0, _CH)],
                                  out_sems[b]),
            pltpu.make_async_copy(o2_b.at[b], out2_hbm.at[:, pl.ds(idx0, _CH)],
                                  out_sems[b]),
        )

    def compute(b):
        @plsc.parallel_loop(0, _GRPS * 16, step=16, unroll=4)
        def g_body(g0):
            iv = idx_b[b, pl.ds(g0, 16)]
            iv2 = iv ^ 1
            for c in range(_C):
                cc = jnp.full((16,), c, jnp.int32)
                w1 = plsc.load_gather(in_b.at[b], [cc, iv])
                w2 = plsc.load_gather(in_b.at[b], [cc, iv2])
                o1_b[b, c, pl.ds(g0, 16)] = w1
                o2_b[b, c, pl.ds(g0, 16)] = w2

    @pl.when(wid < _NCHUNK)
    def _():
        for cp in in_copies(0, 0):
            cp.start()

    def u_body(u, carry):
        for b in (0, 1):
            t = 2 * u + b
            ci = wid + _NW * t

            @pl.when(ci < _NCHUNK)
            def _(t=t, b=b):
                for cp in in_copies(t, b):
                    cp.wait()

            @pl.when(wid + _NW * (t + 1) < _NCHUNK)
            def _(t=t, b=b):
                for cp in in_copies(t + 1, 1 - b):
                    cp.start()

            @pl.when((t >= 2) & (wid + _NW * (t - 2) < _NCHUNK))
            def _(t=t, b=b):
                for cp in out_copies(t - 2, b):
                    cp.wait()

            @pl.when(ci < _NCHUNK)
            def _(t=t, b=b):
                compute(b)
                for cp in out_copies(t, b):
                    cp.start()

        return carry

    lax.fori_loop(0, _UMAX, u_body, 0)


_sc_call = pl.kernel(
    _sc_body,
    out_type=(
        jax.ShapeDtypeStruct((_C, _SL), jnp.float32),
        jax.ShapeDtypeStruct((_C, _SL), jnp.float32),
    ),
    mesh=plsc.VectorSubcoreMesh(core_axis_name="c", subcore_axis_name="s"),
    scratch_types=[
        pltpu.VMEM((2, _C, 2 * _CH), jnp.float32),
        pltpu.VMEM((2, _CH), jnp.int32),
        pltpu.VMEM((2, _C, _CH), jnp.float32),
        pltpu.VMEM((2, _C, _CH), jnp.float32),
        pltpu.SemaphoreType.DMA,
        pltpu.SemaphoreType.DMA,
        pltpu.SemaphoreType.DMA,
        pltpu.SemaphoreType.DMA,
    ],
    compiler_params=pltpu.CompilerParams(needs_layout_passes=False,
                                         disable_bounds_checks=True),
)


def kernel(wav):
    rkey = jax.random.fold_in(jax.random.key(0), 1)
    r = jax.random.randint(rkey, (_SL,), 0, _K)
    lidx = ((jnp.arange(_SL, dtype=jnp.int32) % _CH) * 2 + r).astype(jnp.int32)
    return _sc_call(wav, lidx)


# speedup vs baseline: 12.0752x; 1.1256x over previous
import jax
import jax.numpy as jnp
import numpy as np
from jax import lax
from jax.experimental import pallas as pl
from jax.experimental.pallas import tpu as pltpu
from jax.experimental.pallas import tpu_sc as plsc

_K = 2
_K_CORRECTION = 128

_C = 16
_L = 1048576
_SL = _L // _K - _K_CORRECTION
_CH = 896
_NCHUNK = _SL // _CH
_NW = 32
_ROUNDS = -(-_NCHUNK // _NW)
_GRPS = _CH // 16
_UMAX = (_ROUNDS + 2 + 1) // 2


def _sc_body(wav_hbm, lidx_hbm, out1_hbm, out2_hbm,
             in_b, idx_b, o1_b, o2_b,
             in_sem0, in_sem1, out_sem0, out_sem1):
    wid = lax.axis_index("s") * 2 + lax.axis_index("c")
    in_sems = (in_sem0, in_sem1)
    out_sems = (out_sem0, out_sem1)

    def in_copies(t, b):
        ci = wid + _NW * t
        col0 = pl.multiple_of(ci * (2 * _CH), 2 * _CH)
        idx0 = pl.multiple_of(ci * _CH, _CH)
        return (
            pltpu.make_async_copy(wav_hbm.at[:, pl.ds(col0, 2 * _CH)],
                                  in_b.at[b], in_sems[b]),
            pltpu.make_async_copy(lidx_hbm.at[pl.ds(idx0, _CH)],
                                  idx_b.at[b], in_sems[b]),
        )

    def out_copies(t, b):
        ci = wid + _NW * t
        idx0 = pl.multiple_of(ci * _CH, _CH)
        return (
            pltpu.make_async_copy(o1_b.at[b], out1_hbm.at[:, pl.ds(idx0, _CH)],
                                  out_sems[b]),
            pltpu.make_async_copy(o2_b.at[b], out2_hbm.at[:, pl.ds(idx0, _CH)],
                                  out_sems[b]),
        )

    def compute(b):
        @plsc.parallel_loop(0, _GRPS * 16, step=16, unroll=4)
        def g_body(g0):
            iv = idx_b[b, pl.ds(g0, 16)]
            iv2 = iv ^ 1
            for c in range(_C):
                cc = jnp.full((16,), c, jnp.int32)
                w1 = plsc.load_gather(in_b.at[b], [cc, iv])
                w2 = plsc.load_gather(in_b.at[b], [cc, iv2])
                o1_b[b, c, pl.ds(g0, 16)] = w1
                o2_b[b, c, pl.ds(g0, 16)] = w2

    @pl.when(wid < _NCHUNK)
    def _():
        for cp in in_copies(0, 0):
            cp.start()

    def u_body(u, carry):
        for b in (0, 1):
            t = 2 * u + b
            ci = wid + _NW * t

            @pl.when(ci < _NCHUNK)
            def _(t=t, b=b):
                for cp in in_copies(t, b):
                    cp.wait()

            @pl.when(wid + _NW * (t + 1) < _NCHUNK)
            def _(t=t, b=b):
                for cp in in_copies(t + 1, 1 - b):
                    cp.start()

            @pl.when((t >= 2) & (wid + _NW * (t - 2) < _NCHUNK))
            def _(t=t, b=b):
                for cp in out_copies(t - 2, b):
                    cp.wait()

            @pl.when(ci < _NCHUNK)
            def _(t=t, b=b):
                compute(b)
                for cp in out_copies(t, b):
                    cp.start()

        return carry

    lax.fori_loop(0, _UMAX, u_body, 0)


_sc_call = pl.kernel(
    _sc_body,
    out_type=(
        jax.ShapeDtypeStruct((_C, _SL), jnp.float32),
        jax.ShapeDtypeStruct((_C, _SL), jnp.float32),
    ),
    mesh=plsc.VectorSubcoreMesh(core_axis_name="c", subcore_axis_name="s"),
    scratch_types=[
        pltpu.VMEM((2, _C, 2 * _CH), jnp.float32),
        pltpu.VMEM((2, _CH), jnp.int32),
        pltpu.VMEM((2, _C, _CH), jnp.float32),
        pltpu.VMEM((2, _C, _CH), jnp.float32),
        pltpu.SemaphoreType.DMA,
        pltpu.SemaphoreType.DMA,
        pltpu.SemaphoreType.DMA,
        pltpu.SemaphoreType.DMA,
    ],
    compiler_params=pltpu.CompilerParams(needs_layout_passes=False,
                                         disable_bounds_checks=True),
)


def _make_lidx() -> np.ndarray:
    rkey = jax.random.fold_in(jax.random.key(0), 1)
    r = np.asarray(jax.random.randint(rkey, (_SL,), 0, _K))
    return ((np.arange(_SL) % _CH) * 2 + r).astype(np.int32)


_LIDX = _make_lidx()


def kernel(wav):
    return _sc_call(wav, jnp.asarray(_LIDX))


# speedup vs baseline: 12.1501x; 1.0062x over previous
import jax
import jax.numpy as jnp
import numpy as np
from jax import lax
from jax.experimental import pallas as pl
from jax.experimental.pallas import tpu as pltpu
from jax.experimental.pallas import tpu_sc as plsc

_K = 2
_K_CORRECTION = 128

_C = 16
_L = 1048576
_SL = _L // _K - _K_CORRECTION
_CH = 896
_NCHUNK = _SL // _CH
_NW = 32
_ROUNDS = -(-_NCHUNK // _NW)
_GRPS = _CH // 16
_UMAX = (_ROUNDS + 2 + 1) // 2


def _sc_body(wav_hbm, lidx_hbm, out1_hbm, out2_hbm,
             in_b, idx_b, o1_b, o2_b,
             in_sem0, in_sem1, out_sem0, out_sem1):
    wid = lax.axis_index("s") * 2 + lax.axis_index("c")
    in_sems = (in_sem0, in_sem1)
    out_sems = (out_sem0, out_sem1)

    def in_copies(t, b):
        ci = wid + _NW * t
        col0 = pl.multiple_of(ci * (2 * _CH), 2 * _CH)
        idx0 = pl.multiple_of(ci * _CH, _CH)
        return (
            pltpu.make_async_copy(wav_hbm.at[:, pl.ds(col0, 2 * _CH)],
                                  in_b.at[b], in_sems[b]),
            pltpu.make_async_copy(lidx_hbm.at[pl.ds(idx0, _CH)],
                                  idx_b.at[b], in_sems[b]),
        )

    def out_copies(t, b):
        ci = wid + _NW * t
        idx0 = pl.multiple_of(ci * _CH, _CH)
        return (
            pltpu.make_async_copy(o1_b.at[b], out1_hbm.at[:, pl.ds(idx0, _CH)],
                                  out_sems[b]),
            pltpu.make_async_copy(o2_b.at[b], out2_hbm.at[:, pl.ds(idx0, _CH)],
                                  out_sems[b]),
        )

    def compute(b):
        @plsc.parallel_loop(0, _GRPS * 16, step=16, unroll=4)
        def g_body(g0):
            iv = idx_b[b, pl.ds(g0, 16)]
            iv2 = iv ^ 1
            for c in range(_C):
                cc = jnp.full((16,), c, jnp.int32)
                w1 = plsc.load_gather(in_b.at[b], [cc, iv])
                w2 = plsc.load_gather(in_b.at[b], [cc, iv2])
                o1_b[b, c, pl.ds(g0, 16)] = w1
                o2_b[b, c, pl.ds(g0, 16)] = w2

    @pl.when(wid < _NCHUNK)
    def _():
        for cp in in_copies(0, 0):
            cp.start()

    def u_body(u, carry):
        for b in (0, 1):
            t = 2 * u + b
            ci = wid + _NW * t

            @pl.when(ci < _NCHUNK)
            def _(t=t, b=b):
                for cp in in_copies(t, b):
                    cp.wait()

            @pl.when(wid + _NW * (t + 1) < _NCHUNK)
            def _(t=t, b=b):
                for cp in in_copies(t + 1, 1 - b):
                    cp.start()

            @pl.when((t >= 2) & (wid + _NW * (t - 2) < _NCHUNK))
            def _(t=t, b=b):
                for cp in out_copies(t - 2, b):
                    cp.wait()

            @pl.when(ci < _NCHUNK)
            def _(t=t, b=b):
                compute(b)
                for cp in out_copies(t, b):
                    cp.start()

        return carry

    lax.fori_loop(0, _UMAX, u_body, 0)


_sc_call = pl.kernel(
    _sc_body,
    out_type=(
        jax.ShapeDtypeStruct((_C, _SL), jnp.float32),
        jax.ShapeDtypeStruct((_C, _SL), jnp.float32),
    ),
    mesh=plsc.VectorSubcoreMesh(core_axis_name="c", subcore_axis_name="s"),
    scratch_types=[
        pltpu.VMEM((2, _C, 2 * _CH), jnp.float32),
        pltpu.VMEM((2, _CH), jnp.int32),
        pltpu.VMEM((2, _C, _CH), jnp.float32),
        pltpu.VMEM((2, _C, _CH), jnp.float32),
        pltpu.SemaphoreType.DMA,
        pltpu.SemaphoreType.DMA,
        pltpu.SemaphoreType.DMA,
        pltpu.SemaphoreType.DMA,
    ],
    compiler_params=pltpu.CompilerParams(needs_layout_passes=False,
                                         disable_bounds_checks=True),
)


_U32 = np.uint32


def _threefry2x32(kpair, x0, x1):
    k0, k1 = _U32(kpair[0]), _U32(kpair[1])
    ks = [k0, k1, _U32(k0 ^ k1 ^ _U32(0x1BD11BDA))]
    rotations = [(13, 15, 26, 6), (17, 29, 16, 24)]
    x0 = (x0 + ks[0]).astype(_U32)
    x1 = (x1 + ks[1]).astype(_U32)
    for i in range(5):
        for r in rotations[i % 2]:
            x0 = (x0 + x1).astype(_U32)
            x1 = ((x1 << _U32(r)) | (x1 >> _U32(32 - r))).astype(_U32)
            x1 = (x1 ^ x0).astype(_U32)
        x0 = (x0 + ks[(i + 1) % 3]).astype(_U32)
        x1 = (x1 + ks[(i + 2) % 3] + _U32(i + 1)).astype(_U32)
    return x0, x1


def _make_lidx() -> np.ndarray:
    f0, f1 = _threefry2x32((_U32(0), _U32(0)), np.zeros(1, _U32),
                           np.ones(1, _U32))
    fkey = (f0[0], f1[0])
    s0, s1 = _threefry2x32(fkey, np.zeros(2, _U32), np.arange(2, dtype=_U32))
    k2 = (s0[1], s1[1])
    b0, b1 = _threefry2x32(k2, np.zeros(_SL, _U32), np.arange(_SL, dtype=_U32))
    r = ((b0 ^ b1) % _U32(2)).astype(np.int64)
    return ((np.arange(_SL) % _CH) * 2 + r).astype(np.int32)


_LIDX = _make_lidx()


def kernel(wav):
    return _sc_call(wav, jnp.asarray(_LIDX))
